# Initial kernel scaffold; baseline (speedup 1.0000x reference)
import jax
import jax.numpy as jnp
from jax.experimental import pallas as pl


def kernel(x_global, per_feature_x, river_edge_index, river_edge_attr, causal_edge_index, causal_edge_weight, r_lin_W, r_lin_b, r_upd_W, r_upd_b, r_enc_W, r_enc_b, r_gate, c_lin_W, c_lin_b, c_upd_W, c_upd_b, c_gate, conv_W, conv_b, sf_W, sf_b, feat_emb, pf_g, pf_b, fu_W1, fu_b1, fu_ln_g, fu_ln_b, fu_W2, fu_b2):
    raise NotImplementedError("write your pallas kernel here")


# speedup vs baseline: 1.1220x; 1.1220x over previous
import functools

import jax
import jax.numpy as jnp
import numpy as np
from jax.experimental import pallas as pl
from jax.experimental.pallas import tpu as pltpu

B, N, H, T, F = 2, 1000, 64, 16, 8
Er, Ec = 8000, 128000
BT = B * T
NF = N * F


def _pe_const():
    pe = np.zeros((T, H), np.float32)
    pos = np.arange(T, dtype=np.float32)[:, None]
    div = np.exp(np.arange(0, H, 2, dtype=np.float32) * (-np.log(10000.0) / H))
    pe[:, 0::2] = np.sin(pos * div)
    pe[:, 1::2] = np.cos(pos * div)
    return jnp.asarray(pe)


def _silu(x):
    return x * jax.nn.sigmoid(x)


def _lrelu(x):
    return jnp.where(x >= 0, x, 0.01 * x)


def _edge_w_body(attr4_ref, cew_ref, rencW_ref, scal_ref, mwr_ref, mwc_ref):
    w = rencW_ref[...]
    attr4 = attr4_ref[...]
    ew = (attr4 * w[:, :1]).sum(0, keepdims=True) + scal_ref[0, :1]
    mwr_ref[...] = jnp.clip(scal_ref[1, :1] * ew, 0.0, 1.0)
    mwc_ref[...] = jnp.clip(scal_ref[2, 0] * cew_ref[...], 0.0, 1.0)


def _edge_weights(river_edge_attr, causal_edge_weight, r_enc_W, r_enc_b,
                  r_gate, c_gate):
    attr4 = river_edge_attr.T
    cew = causal_edge_weight.reshape(Ec // 128, 128)
    scal = jnp.stack([
        jnp.broadcast_to(r_enc_b[0], (128,)),
        jnp.broadcast_to(jax.nn.sigmoid(r_gate[0]), (128,)),
        jnp.broadcast_to(jax.nn.sigmoid(c_gate[0]), (128,)),
    ])
    rencW = jnp.broadcast_to(r_enc_W[0][:, None], (4, 128))
    mwr, mwc = pl.pallas_call(
        _edge_w_body,
        out_shape=(jax.ShapeDtypeStruct((1, Er), jnp.float32),
                   jax.ShapeDtypeStruct((Ec // 128, 128), jnp.float32)),
    )(attr4, cew, rencW, scal)
    return mwr[0], mwc.reshape(Ec)


def _prep_body(xg_ref, fnx_ref, rW_ref, rb_ref, cW_ref, cb_ref, emb_ref,
               yr_ref, yc_ref):
    xg = xg_ref[0]
    yr_ref[0] = jnp.dot(xg, rW_ref[...], preferred_element_type=jnp.float32) \
        + rb_ref[...]
    fn = (fnx_ref[0].reshape(N, F, H) + emb_ref[...][None]).reshape(NF, H)
    yc_ref[0] = jnp.dot(fn, cW_ref[...], preferred_element_type=jnp.float32) \
        + cb_ref[...]


def _prep(xg, fnx, r_lin_W, r_lin_b, c_lin_W, c_lin_b, feat_emb):
    return pl.pallas_call(
        _prep_body,
        grid=(BT,),
        in_specs=[
            pl.BlockSpec((1, N, H), lambda s: (s, 0, 0)),
            pl.BlockSpec((1, NF, H), lambda s: (s, 0, 0)),
            pl.BlockSpec((H, H), lambda s: (0, 0)),
            pl.BlockSpec((1, H), lambda s: (0, 0)),
            pl.BlockSpec((H, H), lambda s: (0, 0)),
            pl.BlockSpec((1, H), lambda s: (0, 0)),
            pl.BlockSpec((F, H), lambda s: (0, 0)),
        ],
        out_specs=[
            pl.BlockSpec((1, N, H), lambda s: (s, 0, 0)),
            pl.BlockSpec((1, NF, H), lambda s: (s, 0, 0)),
        ],
        out_shape=[jax.ShapeDtypeStruct((BT, N, H), jnp.float32),
                   jax.ShapeDtypeStruct((BT, NF, H), jnp.float32)],
    )(xg, fnx, r_lin_W.T, r_lin_b[None], c_lin_W.T, c_lin_b[None], feat_emb)


def _post_body(ar_ref, ac_ref, xg_ref, fnx_ref, rA_ref, rB_ref, rb_ref,
               cA_ref, cB_ref, cb_ref, sA_ref, sB_ref, sb_ref, emb_ref,
               png_ref, pnb_ref, fused_ref, pfln_ref):
    f32 = jnp.float32
    w = _lrelu(jnp.dot(ar_ref[0], rA_ref[...], preferred_element_type=f32)
               + jnp.dot(xg_ref[0], rB_ref[...], preferred_element_type=f32)
               + rb_ref[...])
    fn = (fnx_ref[0].reshape(N, F, H) + emb_ref[...][None]).reshape(NF, H)
    fu = _lrelu(jnp.dot(ac_ref[0], cA_ref[...], preferred_element_type=f32)
                + jnp.dot(fn, cB_ref[...], preferred_element_type=f32)
                + cb_ref[...])
    fu4 = fu.reshape(N, F, H)
    pooled = fu4.mean(1)
    fused_ref[0] = _silu(jnp.dot(w, sA_ref[...], preferred_element_type=f32)
                         + jnp.dot(pooled, sB_ref[...],
                                   preferred_element_type=f32)
                         + sb_ref[...])
    m = fu.mean(-1, keepdims=True)
    v = ((fu - m) ** 2).mean(-1, keepdims=True)
    pfln_ref[0] = png_ref[...] * (fu - m) * jax.lax.rsqrt(v + 1e-5) \
        + pnb_ref[...]


def _post(aggr_r, aggr_c, xg, fnx, r_upd_W, r_upd_b, c_upd_W, c_upd_b,
          sf_W, sf_b, feat_emb, pf_g, pf_b):
    return pl.pallas_call(
        _post_body,
        grid=(BT,),
        in_specs=[
            pl.BlockSpec((1, N, H), lambda s: (s, 0, 0)),
            pl.BlockSpec((1, NF, H), lambda s: (s, 0, 0)),
            pl.BlockSpec((1, N, H), lambda s: (s, 0, 0)),
            pl.BlockSpec((1, NF, H), lambda s: (s, 0, 0)),
        ] + [pl.BlockSpec(None, lambda s: (0, 0))] * 12,
        out_specs=[
            pl.BlockSpec((1, N, H), lambda s: (s, 0, 0)),
            pl.BlockSpec((1, NF, H), lambda s: (s, 0, 0)),
        ],
        out_shape=[jax.ShapeDtypeStruct((BT, N, H), jnp.float32),
                   jax.ShapeDtypeStruct((BT, NF, H), jnp.float32)],
    )(aggr_r, aggr_c, xg, fnx,
      r_upd_W[:, :H].T, r_upd_W[:, H:].T, r_upd_b[None],
      c_upd_W[:, :H].T, c_upd_W[:, H:].T, c_upd_b[None],
      sf_W[:, :H].T, sf_W[:, H:].T, sf_b[None],
      feat_emb, pf_g[None], pf_b[None])


def _final_body(xt_ref, xs_ref, pe_ref, W0_ref, W1_ref, W2_ref, cb_ref,
                F1a_ref, F1b_ref, f1b_ref, lg_ref, lb_ref, F2_ref, f2b_ref,
                out_ref):
    f32 = jnp.float32
    C = xt_ref.shape[0]
    x = xt_ref[...] + pe_ref[...][None]
    zero = jnp.zeros((C, 1, H), f32)
    xm = jnp.concatenate([zero, x[:, :-1]], axis=1)
    xp = jnp.concatenate([x[:, 1:], zero], axis=1)
    x2 = x.reshape(C * T, H)
    xtemp = (jnp.dot(xm.reshape(C * T, H), W0_ref[...], preferred_element_type=f32)
             + jnp.dot(x2, W1_ref[...], preferred_element_type=f32)
             + jnp.dot(xp.reshape(C * T, H), W2_ref[...], preferred_element_type=f32)
             + cb_ref[...])
    h = (jnp.dot(xtemp, F1a_ref[...], preferred_element_type=f32)
         + jnp.dot(xs_ref[...].reshape(C * T, H), F1b_ref[...],
                   preferred_element_type=f32)
         + f1b_ref[...])
    m = h.mean(-1, keepdims=True)
    v = ((h - m) ** 2).mean(-1, keepdims=True)
    h = _silu(lg_ref[...] * (h - m) * jax.lax.rsqrt(v + 1e-5) + lb_ref[...])
    out = jnp.dot(h, F2_ref[...], preferred_element_type=f32) + f2b_ref[...]
    out_ref[...] = out.reshape(C, T, H)


def _final(xtp, xs, conv_W, conv_b, fu_W1, fu_b1, fu_ln_g, fu_ln_b,
           fu_W2, fu_b2):
    CH = 500
    return pl.pallas_call(
        _final_body,
        grid=(B * N // CH,),
        in_specs=[
            pl.BlockSpec((CH, T, H), lambda i: (i, 0, 0)),
            pl.BlockSpec((CH, T, H), lambda i: (i, 0, 0)),
        ] + [pl.BlockSpec(None, lambda i: (0, 0))] * 12,
        out_specs=pl.BlockSpec((CH, T, H), lambda i: (i, 0, 0)),
        out_shape=jax.ShapeDtypeStruct((B * N, T, H), jnp.float32),
    )(xtp, xs, _pe_const(),
      conv_W[:, :, 0].T, conv_W[:, :, 1].T, conv_W[:, :, 2].T, conv_b[None],
      fu_W1[:, :H].T, fu_W1[:, H:].T, fu_b1[None],
      fu_ln_g[None], fu_ln_b[None], fu_W2.T, fu_b2[None])


def _aggregate(y, mw, src, dst, nseg):
    msg = mw[None, :, None] * y[:, src]
    return jax.vmap(
        lambda m: jax.ops.segment_sum(m, dst, num_segments=nseg))(msg)


def kernel(x_global, per_feature_x, river_edge_index, river_edge_attr,
           causal_edge_index, causal_edge_weight, r_lin_W, r_lin_b, r_upd_W,
           r_upd_b, r_enc_W, r_enc_b, r_gate, c_lin_W, c_lin_b, c_upd_W,
           c_upd_b, c_gate, conv_W, conv_b, sf_W, sf_b, feat_emb, pf_g, pf_b,
           fu_W1, fu_b1, fu_ln_g, fu_ln_b, fu_W2, fu_b2):
    xg = jnp.transpose(x_global, (0, 3, 1, 2)).reshape(BT, N, H)
    fnx = jnp.transpose(per_feature_x, (0, 4, 1, 2, 3)).reshape(BT, NF, H)

    mw_r, mw_c = _edge_weights(river_edge_attr, causal_edge_weight,
                               r_enc_W, r_enc_b, r_gate, c_gate)
    y_r, y_c = _prep(xg, fnx, r_lin_W, r_lin_b, c_lin_W, c_lin_b, feat_emb)

    aggr_r = _aggregate(y_r, mw_r, river_edge_index[0], river_edge_index[1], N)
    aggr_c = _aggregate(y_c, mw_c, causal_edge_index[0], causal_edge_index[1],
                        NF)

    fused, pfln = _post(aggr_r, aggr_c, xg, fnx, r_upd_W, r_upd_b,
                        c_upd_W, c_upd_b, sf_W, sf_b, feat_emb, pf_g, pf_b)

    pfu = jnp.transpose(pfln.reshape(B, T, N, F, H), (0, 2, 3, 4, 1))

    xtp = jnp.transpose(x_global, (0, 1, 3, 2)).reshape(B * N, T, H)
    xs = jnp.transpose(fused.reshape(B, T, N, H), (0, 2, 1, 3)) \
        .reshape(B * N, T, H)
    out = _final(xtp, xs, conv_W, conv_b, fu_W1, fu_b1, fu_ln_g, fu_ln_b,
                 fu_W2, fu_b2)
    x_out = jnp.transpose(out.reshape(B, N, T, H), (0, 1, 3, 2))
    return (x_out, pfu)


# speedup vs baseline: 11.6938x; 10.4226x over previous
import functools

import jax
import jax.numpy as jnp
import numpy as np
from jax import lax
from jax.experimental import pallas as pl
from jax.experimental.pallas import tpu as pltpu
from jax.experimental.pallas import tpu_sc as plsc

B, N, H, T, F = 2, 1000, 64, 16, 8
Er, Ec = 8000, 128000
BT = B * T
NF = N * F


def _pe_const():
    pe = np.zeros((T, H), np.float32)
    pos = np.arange(T, dtype=np.float32)[:, None]
    div = np.exp(np.arange(0, H, 2, dtype=np.float32) * (-np.log(10000.0) / H))
    pe[:, 0::2] = np.sin(pos * div)
    pe[:, 1::2] = np.cos(pos * div)
    return jnp.asarray(pe)


def _silu(x):
    return x * jax.nn.sigmoid(x)


def _lrelu(x):
    return jnp.where(x >= 0, x, 0.01 * x)


def _edge_w_body(attr4_ref, cew_ref, rencW_ref, scal_ref, mwr_ref, mwc_ref):
    w = rencW_ref[...]
    attr4 = attr4_ref[...]
    ew = (attr4 * w[:, :1]).sum(0, keepdims=True) + scal_ref[0, :1]
    mwr_ref[...] = jnp.clip(scal_ref[1, :1] * ew, 0.0, 1.0)
    mwc_ref[...] = jnp.clip(scal_ref[2, 0] * cew_ref[...], 0.0, 1.0)


def _edge_weights(river_edge_attr, causal_edge_weight, r_enc_W, r_enc_b,
                  r_gate, c_gate):
    attr4 = river_edge_attr.T
    cew = causal_edge_weight.reshape(Ec // 128, 128)
    scal = jnp.stack([
        jnp.broadcast_to(r_enc_b[0], (128,)),
        jnp.broadcast_to(jax.nn.sigmoid(r_gate[0]), (128,)),
        jnp.broadcast_to(jax.nn.sigmoid(c_gate[0]), (128,)),
    ])
    rencW = jnp.broadcast_to(r_enc_W[0][:, None], (4, 128))
    mwr, mwc = pl.pallas_call(
        _edge_w_body,
        out_shape=(jax.ShapeDtypeStruct((1, Er), jnp.float32),
                   jax.ShapeDtypeStruct((Ec // 128, 128), jnp.float32)),
    )(attr4, cew, rencW, scal)
    return mwr[0], mwc.reshape(Ec)


def _prep_body(xg_ref, fnx_ref, rW_ref, rb_ref, cW_ref, cb_ref, emb_ref,
               yr_ref, yc_ref):
    f32 = jnp.float32
    ya = jnp.dot(xg_ref[0], rW_ref[...], preferred_element_type=f32) \
        + rb_ref[...]
    yb = jnp.dot(xg_ref[1], rW_ref[...], preferred_element_type=f32) \
        + rb_ref[...]
    yr_ref[0] = jnp.concatenate([ya, yb], axis=-1)
    fna = (fnx_ref[0].reshape(N, F, H) + emb_ref[...][None]).reshape(NF, H)
    fnb = (fnx_ref[1].reshape(N, F, H) + emb_ref[...][None]).reshape(NF, H)
    ca = jnp.dot(fna, cW_ref[...], preferred_element_type=f32) + cb_ref[...]
    cb = jnp.dot(fnb, cW_ref[...], preferred_element_type=f32) + cb_ref[...]
    yc_ref[0] = jnp.concatenate([ca, cb], axis=-1)


def _prep(xg, fnx, r_lin_W, r_lin_b, c_lin_W, c_lin_b, feat_emb):
    return pl.pallas_call(
        _prep_body,
        grid=(BT // 2,),
        in_specs=[
            pl.BlockSpec((2, N, H), lambda p: (p, 0, 0)),
            pl.BlockSpec((2, NF, H), lambda p: (p, 0, 0)),
            pl.BlockSpec((H, H), lambda p: (0, 0)),
            pl.BlockSpec((1, H), lambda p: (0, 0)),
            pl.BlockSpec((H, H), lambda p: (0, 0)),
            pl.BlockSpec((1, H), lambda p: (0, 0)),
            pl.BlockSpec((F, H), lambda p: (0, 0)),
        ],
        out_specs=[
            pl.BlockSpec((1, N, 2 * H), lambda p: (p, 0, 0)),
            pl.BlockSpec((1, NF, 2 * H), lambda p: (p, 0, 0)),
        ],
        out_shape=[jax.ShapeDtypeStruct((BT // 2, N, 2 * H), jnp.float32),
                   jax.ShapeDtypeStruct((BT // 2, NF, 2 * H), jnp.float32)],
    )(xg, fnx, r_lin_W.T, r_lin_b[None], c_lin_W.T, c_lin_b[None], feat_emb)


def _post_body(ar_ref, ac_ref, xg_ref, fnx_ref, rA_ref, rB_ref, rb_ref,
               cA_ref, cB_ref, cb_ref, sA_ref, sB_ref, sb_ref, emb_ref,
               png_ref, pnb_ref, fused_ref, pfln_ref):
    f32 = jnp.float32
    for q in range(2):
        ar = ar_ref[0][:N, q * H:(q + 1) * H]
        ac = ac_ref[0][:NF, q * H:(q + 1) * H]
        w = _lrelu(jnp.dot(ar, rA_ref[...], preferred_element_type=f32)
                   + jnp.dot(xg_ref[q], rB_ref[...],
                             preferred_element_type=f32)
                   + rb_ref[...])
        fn = (fnx_ref[q].reshape(N, F, H) + emb_ref[...][None]).reshape(NF, H)
        fu = _lrelu(jnp.dot(ac, cA_ref[...], preferred_element_type=f32)
                    + jnp.dot(fn, cB_ref[...], preferred_element_type=f32)
                    + cb_ref[...])
        fu4 = fu.reshape(N, F, H)
        pooled = fu4.mean(1)
        fused_ref[q] = _silu(jnp.dot(w, sA_ref[...],
                                     preferred_element_type=f32)
                             + jnp.dot(pooled, sB_ref[...],
                                       preferred_element_type=f32)
                             + sb_ref[...])
        m = fu.mean(-1, keepdims=True)
        v = ((fu - m) ** 2).mean(-1, keepdims=True)
        pfln_ref[q] = png_ref[...] * (fu - m) * jax.lax.rsqrt(v + 1e-5) \
            + pnb_ref[...]


def _post(aggr_r, aggr_c, xg, fnx, r_upd_W, r_upd_b, c_upd_W, c_upd_b,
          sf_W, sf_b, feat_emb, pf_g, pf_b):
    return pl.pallas_call(
        _post_body,
        grid=(BT // 2,),
        in_specs=[
            pl.BlockSpec((1, NP, 2 * H), lambda p: (p, 0, 0)),
            pl.BlockSpec((1, NFP, 2 * H), lambda p: (p, 0, 0)),
            pl.BlockSpec((2, N, H), lambda p: (p, 0, 0)),
            pl.BlockSpec((2, NF, H), lambda p: (p, 0, 0)),
        ] + [pl.BlockSpec(None, lambda p: (0, 0))] * 12,
        out_specs=[
            pl.BlockSpec((2, N, H), lambda p: (p, 0, 0)),
            pl.BlockSpec((2, NF, H), lambda p: (p, 0, 0)),
        ],
        out_shape=[jax.ShapeDtypeStruct((BT, N, H), jnp.float32),
                   jax.ShapeDtypeStruct((BT, NF, H), jnp.float32)],
    )(aggr_r, aggr_c, xg, fnx,
      r_upd_W[:, :H].T, r_upd_W[:, H:].T, r_upd_b[None],
      c_upd_W[:, :H].T, c_upd_W[:, H:].T, c_upd_b[None],
      sf_W[:, :H].T, sf_W[:, H:].T, sf_b[None],
      feat_emb, pf_g[None], pf_b[None])


def _final_body(xt_ref, xs_ref, pe_ref, W0_ref, W1_ref, W2_ref, cb_ref,
                F1a_ref, F1b_ref, f1b_ref, lg_ref, lb_ref, F2_ref, f2b_ref,
                out_ref):
    f32 = jnp.float32
    C = xt_ref.shape[0]
    x = xt_ref[...] + pe_ref[...][None]
    zero = jnp.zeros((C, 1, H), f32)
    xm = jnp.concatenate([zero, x[:, :-1]], axis=1)
    xp = jnp.concatenate([x[:, 1:], zero], axis=1)
    x2 = x.reshape(C * T, H)
    xtemp = (jnp.dot(xm.reshape(C * T, H), W0_ref[...], preferred_element_type=f32)
             + jnp.dot(x2, W1_ref[...], preferred_element_type=f32)
             + jnp.dot(xp.reshape(C * T, H), W2_ref[...], preferred_element_type=f32)
             + cb_ref[...])
    h = (jnp.dot(xtemp, F1a_ref[...], preferred_element_type=f32)
         + jnp.dot(xs_ref[...].reshape(C * T, H), F1b_ref[...],
                   preferred_element_type=f32)
         + f1b_ref[...])
    m = h.mean(-1, keepdims=True)
    v = ((h - m) ** 2).mean(-1, keepdims=True)
    h = _silu(lg_ref[...] * (h - m) * jax.lax.rsqrt(v + 1e-5) + lb_ref[...])
    out = jnp.dot(h, F2_ref[...], preferred_element_type=f32) + f2b_ref[...]
    out_ref[...] = out.reshape(C, T, H)


def _final(xtp, xs, conv_W, conv_b, fu_W1, fu_b1, fu_ln_g, fu_ln_b,
           fu_W2, fu_b2):
    CH = 500
    return pl.pallas_call(
        _final_body,
        grid=(B * N // CH,),
        in_specs=[
            pl.BlockSpec((CH, T, H), lambda i: (i, 0, 0)),
            pl.BlockSpec((CH, T, H), lambda i: (i, 0, 0)),
        ] + [pl.BlockSpec(None, lambda i: (0, 0))] * 12,
        out_specs=pl.BlockSpec((CH, T, H), lambda i: (i, 0, 0)),
        out_shape=jax.ShapeDtypeStruct((B * N, T, H), jnp.float32),
    )(xtp, xs, _pe_const(),
      conv_W[:, :, 0].T, conv_W[:, :, 1].T, conv_W[:, :, 2].T, conv_b[None],
      fu_W1[:, :H].T, fu_W1[:, H:].T, fu_b1[None],
      fu_ln_g[None], fu_ln_b[None], fu_W2.T, fu_b2[None])


EC_PT = 8192
ER_PT = 512
CH = 128
NC_C = EC_PT // CH
NC_R = ER_PT // CH
NFP = 8192
NP = 1024
STRIPE_C = NFP // 16
STRIPE_Q = STRIPE_C // 4
H2 = 2 * H
NPAIR = BT // 2
PAIRS_PER_CORE = NPAIR // 2


def _sc_aggr_body(yc_hbm, yr_hbm, srcc_hbm, dstc_hbm, mwc_hbm,
                  srcr_hbm, dstr_hbm, mwr_hbm, outc_hbm, outr_hbm,
                  acc_c, acc_r, srcc, dstc, mwc, srcr, dstr, mwr,
                  idxc, idxr, rows, sem):
    cid = lax.axis_index("c")
    sid = lax.axis_index("s")

    pltpu.sync_copy(srcc_hbm.at[sid], srcc)
    pltpu.sync_copy(dstc_hbm.at[sid], dstc)
    pltpu.sync_copy(mwc_hbm.at[sid], mwc)
    pltpu.sync_copy(srcr_hbm.at[sid], srcr)
    pltpu.sync_copy(dstr_hbm.at[sid], dstr)
    pltpu.sync_copy(mwr_hbm.at[sid], mwr)

    def _pair_step(p_local, _):
        p_glob = cid * PAIRS_PER_CORE + p_local

        def _zero_row(r, _):
            for h in range(H2 // 16):
                rows[r, pl.ds(h * 16, 16)] = jnp.zeros((16,), jnp.float32)
            return _
        lax.fori_loop(0, CH, _zero_row, None)
        for q in range(4):
            pltpu.sync_copy(
                rows, acc_c.at[pl.ds(sid * STRIPE_C + q * STRIPE_Q, STRIPE_Q)])
        pltpu.sync_copy(rows.at[pl.ds(0, 64)], acc_r.at[pl.ds(sid * 64, 64)])

        offc = p_glob * NF
        offr = p_glob * N

        def _idx_c(j, _):
            def _v(v, _):
                idxc[j, pl.ds(v * 16, 16)] = srcc[j, pl.ds(v * 16, 16)] + offc
                return _
            return lax.fori_loop(0, CH // 16, _v, _)
        lax.fori_loop(0, NC_C, _idx_c, None)

        def _idx_r(j, _):
            def _v(v, _):
                idxr[j, pl.ds(v * 16, 16)] = srcr[j, pl.ds(v * 16, 16)] + offr
                return _
            return lax.fori_loop(0, CH // 16, _v, _)
        lax.fori_loop(0, NC_R, _idx_r, None)

        plsc.subcore_barrier()

        def _scale_group(mw_ref, j, g):
            mvec = mw_ref[j, pl.ds(g * 16, 16)]
            e0 = g * 16
            for k in range(16):
                m = mvec[k]
                for h in range(H2 // 16):
                    sl = pl.ds(h * 16, 16)
                    rows[e0 + k, sl] = rows[e0 + k, sl] * m

        def _chunk_c(j, _):
            pltpu.async_copy(yc_hbm.at[idxc.at[j]], rows, sem).wait()

            def _g(g, _):
                _scale_group(mwc, j, g)
                return _
            lax.fori_loop(0, CH // 16, _g, None)
            pltpu.sync_copy(rows, acc_c.at[dstc.at[j]], add=True)
            return _
        lax.fori_loop(0, NC_C, _chunk_c, None)

        def _chunk_r(j, _):
            pltpu.async_copy(yr_hbm.at[idxr.at[j]], rows, sem).wait()

            def _g(g, _):
                _scale_group(mwr, j, g)
                return _
            lax.fori_loop(0, CH // 16, _g, None)
            pltpu.sync_copy(rows, acc_r.at[dstr.at[j]], add=True)
            return _
        lax.fori_loop(0, NC_R, _chunk_r, None)

        plsc.subcore_barrier()

        for q in range(4):
            off = sid * STRIPE_C + q * STRIPE_Q
            pltpu.sync_copy(acc_c.at[pl.ds(off, STRIPE_Q)],
                            outc_hbm.at[pl.ds(p_glob * NFP + off, STRIPE_Q)])
        pltpu.sync_copy(acc_r.at[pl.ds(sid * 64, 64)],
                        outr_hbm.at[pl.ds(p_glob * NP + sid * 64, 64)])
        return _

    lax.fori_loop(0, PAIRS_PER_CORE, _pair_step, None)


def _sc_aggregate(y_c, y_r, src_c, dst_c, mw_c, src_r, dst_r, mw_r):
    mesh = plsc.VectorSubcoreMesh(core_axis_name="c", subcore_axis_name="s",
                                  num_cores=2, num_subcores=16)
    f32, i32 = jnp.float32, jnp.int32
    kern = pl.kernel(
        _sc_aggr_body,
        out_type=(jax.ShapeDtypeStruct((NPAIR * NFP, H2), f32),
                  jax.ShapeDtypeStruct((NPAIR * NP, H2), f32)),
        mesh=mesh,
        scratch_types=[
            pltpu.VMEM_SHARED((NFP, H2), f32),
            pltpu.VMEM_SHARED((NP, H2), f32),
            pltpu.VMEM((NC_C, CH), i32),
            pltpu.VMEM((NC_C, CH), i32),
            pltpu.VMEM((NC_C, CH), f32),
            pltpu.VMEM((NC_R, CH), i32),
            pltpu.VMEM((NC_R, CH), i32),
            pltpu.VMEM((NC_R, CH), f32),
            pltpu.VMEM((NC_C, CH), i32),
            pltpu.VMEM((NC_R, CH), i32),
            pltpu.VMEM((CH, H2), f32),
            pltpu.SemaphoreType.DMA,
        ],
    )
    return kern(y_c.reshape(NPAIR * NF, H2), y_r.reshape(NPAIR * N, H2),
                src_c, dst_c, mw_c, src_r, dst_r, mw_r)


def _pad_edges(src, dst, mw, e_real, e_pad, nseg, ntile, nchunk):
    pad = e_pad - e_real
    fill = jnp.arange(pad, dtype=jnp.int32) % nseg
    srcp = jnp.concatenate([src.astype(jnp.int32), fill])
    dstp = jnp.concatenate([dst.astype(jnp.int32), fill])
    mwp = jnp.concatenate([mw, jnp.zeros((pad,), jnp.float32)])
    rs = lambda a: a.reshape(ntile, nchunk, CH)
    return rs(srcp), rs(dstp), rs(mwp)


def kernel(x_global, per_feature_x, river_edge_index, river_edge_attr,
           causal_edge_index, causal_edge_weight, r_lin_W, r_lin_b, r_upd_W,
           r_upd_b, r_enc_W, r_enc_b, r_gate, c_lin_W, c_lin_b, c_upd_W,
           c_upd_b, c_gate, conv_W, conv_b, sf_W, sf_b, feat_emb, pf_g, pf_b,
           fu_W1, fu_b1, fu_ln_g, fu_ln_b, fu_W2, fu_b2):
    xg = jnp.transpose(x_global, (0, 3, 1, 2)).reshape(BT, N, H)
    fnx = jnp.transpose(per_feature_x, (0, 4, 1, 2, 3)).reshape(BT, NF, H)

    mw_r, mw_c = _edge_weights(river_edge_attr, causal_edge_weight,
                               r_enc_W, r_enc_b, r_gate, c_gate)
    y_r, y_c = _prep(xg, fnx, r_lin_W, r_lin_b, c_lin_W, c_lin_b, feat_emb)

    src_c, dst_c, mw_cp = _pad_edges(causal_edge_index[0],
                                     causal_edge_index[1], mw_c,
                                     Ec, 16 * EC_PT, NF, 16, NC_C)
    src_r, dst_r, mw_rp = _pad_edges(river_edge_index[0], river_edge_index[1],
                                     mw_r, Er, 16 * ER_PT, N, 16, NC_R)
    aggr_c_p, aggr_r_p = _sc_aggregate(y_c, y_r, src_c, dst_c, mw_cp,
                                       src_r, dst_r, mw_rp)
    aggr_r = aggr_r_p.reshape(NPAIR, NP, H2)
    aggr_c = aggr_c_p.reshape(NPAIR, NFP, H2)

    fused, pfln = _post(aggr_r, aggr_c, xg, fnx, r_upd_W, r_upd_b,
                        c_upd_W, c_upd_b, sf_W, sf_b, feat_emb, pf_g, pf_b)

    pfu = jnp.transpose(pfln.reshape(B, T, N, F, H), (0, 2, 3, 4, 1))

    xtp = jnp.transpose(x_global, (0, 1, 3, 2)).reshape(B * N, T, H)
    xs = jnp.transpose(fused.reshape(B, T, N, H), (0, 2, 1, 3)) \
        .reshape(B * N, T, H)
    out = _final(xtp, xs, conv_W, conv_b, fu_W1, fu_b1, fu_ln_g, fu_ln_b,
                 fu_W2, fu_b2)
    x_out = jnp.transpose(out.reshape(B, N, T, H), (0, 1, 3, 2))
    return (x_out, pfu)


# speedup vs baseline: 17.1740x; 1.4686x over previous
import functools

import jax
import jax.numpy as jnp
import numpy as np
from jax import lax
from jax.experimental import pallas as pl
from jax.experimental.pallas import tpu as pltpu
from jax.experimental.pallas import tpu_sc as plsc

B, N, H, T, F = 2, 1000, 64, 16, 8
Er, Ec = 8000, 128000
BT = B * T
NF = N * F


def _pe_const():
    pe = np.zeros((T, H), np.float32)
    pos = np.arange(T, dtype=np.float32)[:, None]
    div = np.exp(np.arange(0, H, 2, dtype=np.float32) * (-np.log(10000.0) / H))
    pe[:, 0::2] = np.sin(pos * div)
    pe[:, 1::2] = np.cos(pos * div)
    return jnp.asarray(pe)


def _silu(x):
    return x * jax.nn.sigmoid(x)


def _lrelu(x):
    return jnp.where(x >= 0, x, 0.01 * x)


def _edge_w_body(attr4_ref, cew_ref, rencW_ref, scal_ref, mwr_ref, mwc_ref):
    w = rencW_ref[...]
    attr4 = attr4_ref[...]
    ew = (attr4 * w[:, :1]).sum(0, keepdims=True) + scal_ref[0, :1]
    mwr_ref[...] = jnp.clip(scal_ref[1, :1] * ew, 0.0, 1.0)
    mwc_ref[...] = jnp.clip(scal_ref[2, 0] * cew_ref[...], 0.0, 1.0)


def _edge_weights(river_edge_attr, causal_edge_weight, r_enc_W, r_enc_b,
                  r_gate, c_gate):
    attr4 = river_edge_attr.T
    cew = causal_edge_weight.reshape(Ec // 128, 128)
    scal = jnp.stack([
        jnp.broadcast_to(r_enc_b[0], (128,)),
        jnp.broadcast_to(jax.nn.sigmoid(r_gate[0]), (128,)),
        jnp.broadcast_to(jax.nn.sigmoid(c_gate[0]), (128,)),
    ])
    rencW = jnp.broadcast_to(r_enc_W[0][:, None], (4, 128))
    mwr, mwc = pl.pallas_call(
        _edge_w_body,
        out_shape=(jax.ShapeDtypeStruct((1, Er), jnp.float32),
                   jax.ShapeDtypeStruct((Ec // 128, 128), jnp.float32)),
    )(attr4, cew, rencW, scal)
    return mwr[0], mwc.reshape(Ec)


def _prep_body(xg_ref, fnx_ref, rW_ref, rb_ref, cW_ref, cb_ref, emb_ref,
               yr_ref, yc_ref):
    f32 = jnp.float32
    ya = jnp.dot(xg_ref[0], rW_ref[...], preferred_element_type=f32) \
        + rb_ref[...]
    yb = jnp.dot(xg_ref[1], rW_ref[...], preferred_element_type=f32) \
        + rb_ref[...]
    yr_ref[0] = jnp.concatenate([ya, yb], axis=-1)
    fna = (fnx_ref[0].reshape(N, F, H) + emb_ref[...][None]).reshape(NF, H)
    fnb = (fnx_ref[1].reshape(N, F, H) + emb_ref[...][None]).reshape(NF, H)
    ca = jnp.dot(fna, cW_ref[...], preferred_element_type=f32) + cb_ref[...]
    cb = jnp.dot(fnb, cW_ref[...], preferred_element_type=f32) + cb_ref[...]
    yc_ref[0] = jnp.concatenate([ca, cb], axis=-1)


def _prep(xg, fnx, r_lin_W, r_lin_b, c_lin_W, c_lin_b, feat_emb):
    return pl.pallas_call(
        _prep_body,
        grid=(BT // 2,),
        in_specs=[
            pl.BlockSpec((2, N, H), lambda p: (p, 0, 0)),
            pl.BlockSpec((2, NF, H), lambda p: (p, 0, 0)),
            pl.BlockSpec((H, H), lambda p: (0, 0)),
            pl.BlockSpec((1, H), lambda p: (0, 0)),
            pl.BlockSpec((H, H), lambda p: (0, 0)),
            pl.BlockSpec((1, H), lambda p: (0, 0)),
            pl.BlockSpec((F, H), lambda p: (0, 0)),
        ],
        out_specs=[
            pl.BlockSpec((1, N, 2 * H), lambda p: (p, 0, 0)),
            pl.BlockSpec((1, NF, 2 * H), lambda p: (p, 0, 0)),
        ],
        out_shape=[jax.ShapeDtypeStruct((BT // 2, N, 2 * H), jnp.float32),
                   jax.ShapeDtypeStruct((BT // 2, NF, 2 * H), jnp.float32)],
    )(xg, fnx, r_lin_W.T, r_lin_b[None], c_lin_W.T, c_lin_b[None], feat_emb)


def _post_body(ar_ref, ac_ref, xg_ref, fnx_ref, rA_ref, rB_ref, rb_ref,
               cA_ref, cB_ref, cb_ref, sA_ref, sB_ref, sb_ref, emb_ref,
               png_ref, pnb_ref, fused_ref, pfln_ref):
    f32 = jnp.float32
    for q in range(2):
        ar = ar_ref[0][:N, q * H:(q + 1) * H]
        ac = ac_ref[0][:NF, q * H:(q + 1) * H]
        w = _lrelu(jnp.dot(ar, rA_ref[...], preferred_element_type=f32)
                   + jnp.dot(xg_ref[q], rB_ref[...],
                             preferred_element_type=f32)
                   + rb_ref[...])
        fn = (fnx_ref[q].reshape(N, F, H) + emb_ref[...][None]).reshape(NF, H)
        fu = _lrelu(jnp.dot(ac, cA_ref[...], preferred_element_type=f32)
                    + jnp.dot(fn, cB_ref[...], preferred_element_type=f32)
                    + cb_ref[...])
        fu4 = fu.reshape(N, F, H)
        pooled = fu4.mean(1)
        fused_ref[q] = _silu(jnp.dot(w, sA_ref[...],
                                     preferred_element_type=f32)
                             + jnp.dot(pooled, sB_ref[...],
                                       preferred_element_type=f32)
                             + sb_ref[...])
        m = fu.mean(-1, keepdims=True)
        v = ((fu - m) ** 2).mean(-1, keepdims=True)
        pfln_ref[q] = png_ref[...] * (fu - m) * jax.lax.rsqrt(v + 1e-5) \
            + pnb_ref[...]


def _post(aggr_r, aggr_c, xg, fnx, r_upd_W, r_upd_b, c_upd_W, c_upd_b,
          sf_W, sf_b, feat_emb, pf_g, pf_b):
    return pl.pallas_call(
        _post_body,
        grid=(BT // 2,),
        in_specs=[
            pl.BlockSpec((1, NPA, 2 * H), lambda p: (p, 0, 0)),
            pl.BlockSpec((1, NFP, 2 * H), lambda p: (p, 0, 0)),
            pl.BlockSpec((2, N, H), lambda p: (p, 0, 0)),
            pl.BlockSpec((2, NF, H), lambda p: (p, 0, 0)),
        ] + [pl.BlockSpec(None, lambda p: (0, 0))] * 12,
        out_specs=[
            pl.BlockSpec((2, N, H), lambda p: (p, 0, 0)),
            pl.BlockSpec((2, NF, H), lambda p: (p, 0, 0)),
        ],
        out_shape=[jax.ShapeDtypeStruct((BT, N, H), jnp.float32),
                   jax.ShapeDtypeStruct((BT, NF, H), jnp.float32)],
    )(aggr_r, aggr_c, xg, fnx,
      r_upd_W[:, :H].T, r_upd_W[:, H:].T, r_upd_b[None],
      c_upd_W[:, :H].T, c_upd_W[:, H:].T, c_upd_b[None],
      sf_W[:, :H].T, sf_W[:, H:].T, sf_b[None],
      feat_emb, pf_g[None], pf_b[None])


def _final_body(xt_ref, xs_ref, pe_ref, W0_ref, W1_ref, W2_ref, cb_ref,
                F1a_ref, F1b_ref, f1b_ref, lg_ref, lb_ref, F2_ref, f2b_ref,
                out_ref):
    f32 = jnp.float32
    C = xt_ref.shape[0]
    x = xt_ref[...] + pe_ref[...][None]
    zero = jnp.zeros((C, 1, H), f32)
    xm = jnp.concatenate([zero, x[:, :-1]], axis=1)
    xp = jnp.concatenate([x[:, 1:], zero], axis=1)
    x2 = x.reshape(C * T, H)
    xtemp = (jnp.dot(xm.reshape(C * T, H), W0_ref[...], preferred_element_type=f32)
             + jnp.dot(x2, W1_ref[...], preferred_element_type=f32)
             + jnp.dot(xp.reshape(C * T, H), W2_ref[...], preferred_element_type=f32)
             + cb_ref[...])
    h = (jnp.dot(xtemp, F1a_ref[...], preferred_element_type=f32)
         + jnp.dot(xs_ref[...].reshape(C * T, H), F1b_ref[...],
                   preferred_element_type=f32)
         + f1b_ref[...])
    m = h.mean(-1, keepdims=True)
    v = ((h - m) ** 2).mean(-1, keepdims=True)
    h = _silu(lg_ref[...] * (h - m) * jax.lax.rsqrt(v + 1e-5) + lb_ref[...])
    out = jnp.dot(h, F2_ref[...], preferred_element_type=f32) + f2b_ref[...]
    out_ref[...] = out.reshape(C, T, H)


def _final(xtp, xs, conv_W, conv_b, fu_W1, fu_b1, fu_ln_g, fu_ln_b,
           fu_W2, fu_b2):
    CH = 500
    return pl.pallas_call(
        _final_body,
        grid=(B * N // CH,),
        in_specs=[
            pl.BlockSpec((CH, T, H), lambda i: (i, 0, 0)),
            pl.BlockSpec((CH, T, H), lambda i: (i, 0, 0)),
        ] + [pl.BlockSpec(None, lambda i: (0, 0))] * 12,
        out_specs=pl.BlockSpec((CH, T, H), lambda i: (i, 0, 0)),
        out_shape=jax.ShapeDtypeStruct((B * N, T, H), jnp.float32),
    )(xtp, xs, _pe_const(),
      conv_W[:, :, 0].T, conv_W[:, :, 1].T, conv_W[:, :, 2].T, conv_b[None],
      fu_W1[:, :H].T, fu_W1[:, H:].T, fu_b1[None],
      fu_ln_g[None], fu_ln_b[None], fu_W2.T, fu_b2[None])


EC_PT = 8192
ER_PT = 512
CH = 128
NC_C = EC_PT // CH
NC_R = ER_PT // CH
NFP = 8000
NPA = 1016
H2 = 2 * H
NPAIR = BT // 2
PAIRS_PER_CORE = NPAIR // 2


def _sc_aggr_body(yc_hbm, yr_hbm, srcc_hbm, dstc_hbm, mwc_hbm,
                  srcr_hbm, dstr_hbm, mwr_hbm, outc_hbm, outr_hbm,
                  acc_c, acc_r, dstc, mwc, dstr, mwr,
                  idxc, idxr, rows, sem0, sem1):
    cid = lax.axis_index("c")
    sid = lax.axis_index("s")
    gsems = [sem0, sem1]

    pltpu.sync_copy(srcc_hbm.at[sid], idxc)
    pltpu.sync_copy(dstc_hbm.at[sid], dstc)
    pltpu.sync_copy(mwc_hbm.at[sid], mwc)
    pltpu.sync_copy(srcr_hbm.at[sid], idxr)
    pltpu.sync_copy(dstr_hbm.at[sid], dstr)
    pltpu.sync_copy(mwr_hbm.at[sid], mwr)

    base_c = cid * PAIRS_PER_CORE * NF
    base_r = cid * PAIRS_PER_CORE * N

    def _shift_idx(idx_ref, nchunk, delta):
        def _j(j, _):
            def _v(v, _):
                sl = pl.ds(v * 16, 16)
                idx_ref[j, sl] = idx_ref[j, sl] + delta
                return _
            return lax.fori_loop(0, CH // 16, _v, _)
        lax.fori_loop(0, nchunk, _j, None)

    _shift_idx(idxc, NC_C, base_c)
    _shift_idx(idxr, NC_R, base_r)

    bufs = [rows.at[pl.ds(0, CH)], rows.at[pl.ds(CH, CH)]]

    def _pair_step(p_local, _):
        p_glob = cid * PAIRS_PER_CORE + p_local

        def _zero_row(r, _):
            for h in range(H2 // 16):
                rows[r, pl.ds(h * 16, 16)] = jnp.zeros((16,), jnp.float32)
            return _
        lax.fori_loop(0, CH, _zero_row, None)

        @pl.when(sid < 15)
        def _():
            for q in range(4):
                pltpu.sync_copy(bufs[0],
                                acc_c.at[pl.ds(sid * 512 + q * CH, CH)])
            pltpu.sync_copy(bufs[0].at[pl.ds(0, 64)],
                            acc_r.at[pl.ds(sid * 64, 64)])

        @pl.when(sid == 15)
        def _():
            for q in range(2):
                pltpu.sync_copy(bufs[0], acc_c.at[pl.ds(7680 + q * CH, CH)])
            pltpu.sync_copy(bufs[0].at[pl.ds(0, 64)],
                            acc_c.at[pl.ds(7936, 64)])
            pltpu.sync_copy(bufs[0].at[pl.ds(0, 56)],
                            acc_r.at[pl.ds(960, 56)])

        plsc.subcore_barrier()

        def _scale_chunk(mw_ref, rbuf, j):
            def _g(g, _):
                mvec = mw_ref[j, pl.ds(g * 16, 16)]
                e0 = g * 16
                for k in range(16):
                    m = mvec[k]
                    for h in range(H2 // 16):
                        sl = pl.ds(h * 16, 16)
                        rbuf[e0 + k, sl] = rbuf[e0 + k, sl] * m
                return _
            lax.fori_loop(0, CH // 16, _g, None)

        pltpu.async_copy(yc_hbm.at[idxc.at[0]], bufs[0], gsems[0])

        def _duo_c(i, _):
            for k in range(2):
                j = 2 * i + k
                kn = 1 - k
                @pl.when(j + 1 < NC_C)
                def _():
                    pltpu.async_copy(yc_hbm.at[idxc.at[j + 1]], bufs[kn],
                                     gsems[kn])
                pltpu.make_async_copy(yc_hbm.at[idxc.at[j]], bufs[k],
                                      gsems[k]).wait()
                _scale_chunk(mwc, bufs[k], j)
                pltpu.sync_copy(bufs[k], acc_c.at[dstc.at[j]], add=True)
            return _
        lax.fori_loop(0, NC_C // 2, _duo_c, None)

        pltpu.async_copy(yr_hbm.at[idxr.at[0]], bufs[0], gsems[0])
        for k in range(NC_R):
            kk = k % 2
            if k + 1 < NC_R:
                pltpu.async_copy(yr_hbm.at[idxr.at[k + 1]], bufs[1 - kk],
                                 gsems[1 - kk])
            pltpu.make_async_copy(yr_hbm.at[idxr.at[k]], bufs[kk],
                                  gsems[kk]).wait()
            _scale_chunk(mwr, bufs[kk], k)
            pltpu.sync_copy(bufs[kk], acc_r.at[dstr.at[k]], add=True)

        _shift_idx(idxc, NC_C, NF)
        _shift_idx(idxr, NC_R, N)

        plsc.subcore_barrier()

        @pl.when(sid < 15)
        def _():
            for q in range(4):
                off = sid * 512 + q * CH
                pltpu.sync_copy(acc_c.at[pl.ds(off, CH)],
                                outc_hbm.at[pl.ds(p_glob * NFP + off, CH)])
            pltpu.sync_copy(
                acc_r.at[pl.ds(sid * 64, 64)],
                outr_hbm.at[pl.ds(p_glob * NPA + sid * 64, 64)])

        @pl.when(sid == 15)
        def _():
            for q in range(2):
                off = 7680 + q * CH
                pltpu.sync_copy(acc_c.at[pl.ds(off, CH)],
                                outc_hbm.at[pl.ds(p_glob * NFP + off, CH)])
            pltpu.sync_copy(acc_c.at[pl.ds(7936, 64)],
                            outc_hbm.at[pl.ds(p_glob * NFP + 7936, 64)])
            pltpu.sync_copy(acc_r.at[pl.ds(960, 56)],
                            outr_hbm.at[pl.ds(p_glob * NPA + 960, 56)])
        return _

    lax.fori_loop(0, PAIRS_PER_CORE, _pair_step, None)


def _sc_aggregate(y_c, y_r, src_c, dst_c, mw_c, src_r, dst_r, mw_r):
    mesh = plsc.VectorSubcoreMesh(core_axis_name="c", subcore_axis_name="s",
                                  num_cores=2, num_subcores=16)
    f32, i32 = jnp.float32, jnp.int32
    kern = pl.kernel(
        _sc_aggr_body,
        out_type=(jax.ShapeDtypeStruct((NPAIR * NFP, H2), f32),
                  jax.ShapeDtypeStruct((NPAIR * NPA, H2), f32)),
        mesh=mesh,
        scratch_types=[
            pltpu.VMEM_SHARED((NFP, H2), f32),
            pltpu.VMEM_SHARED((NPA, H2), f32),
            pltpu.VMEM((NC_C, CH), i32),
            pltpu.VMEM((NC_C, CH), f32),
            pltpu.VMEM((NC_R, CH), i32),
            pltpu.VMEM((NC_R, CH), f32),
            pltpu.VMEM((NC_C, CH), i32),
            pltpu.VMEM((NC_R, CH), i32),
            pltpu.VMEM((2 * CH, H2), f32),
            pltpu.SemaphoreType.DMA,
            pltpu.SemaphoreType.DMA,
        ],
    )
    return kern(y_c.reshape(NPAIR * NF, H2), y_r.reshape(NPAIR * N, H2),
                src_c, dst_c, mw_c, src_r, dst_r, mw_r)


def _pad_edges(src, dst, mw, e_real, e_pad, nseg, ntile, nchunk):
    pad = e_pad - e_real
    fill = jnp.arange(pad, dtype=jnp.int32) % nseg
    srcp = jnp.concatenate([src.astype(jnp.int32), fill])
    dstp = jnp.concatenate([dst.astype(jnp.int32), fill])
    mwp = jnp.concatenate([mw, jnp.zeros((pad,), jnp.float32)])
    rs = lambda a: a.reshape(ntile, nchunk, CH)
    return rs(srcp), rs(dstp), rs(mwp)


def kernel(x_global, per_feature_x, river_edge_index, river_edge_attr,
           causal_edge_index, causal_edge_weight, r_lin_W, r_lin_b, r_upd_W,
           r_upd_b, r_enc_W, r_enc_b, r_gate, c_lin_W, c_lin_b, c_upd_W,
           c_upd_b, c_gate, conv_W, conv_b, sf_W, sf_b, feat_emb, pf_g, pf_b,
           fu_W1, fu_b1, fu_ln_g, fu_ln_b, fu_W2, fu_b2):
    xg = jnp.transpose(x_global, (0, 3, 1, 2)).reshape(BT, N, H)
    fnx = jnp.transpose(per_feature_x, (0, 4, 1, 2, 3)).reshape(BT, NF, H)

    mw_r, mw_c = _edge_weights(river_edge_attr, causal_edge_weight,
                               r_enc_W, r_enc_b, r_gate, c_gate)
    y_r, y_c = _prep(xg, fnx, r_lin_W, r_lin_b, c_lin_W, c_lin_b, feat_emb)

    src_c, dst_c, mw_cp = _pad_edges(causal_edge_index[0],
                                     causal_edge_index[1], mw_c,
                                     Ec, 16 * EC_PT, NF, 16, NC_C)
    src_r, dst_r, mw_rp = _pad_edges(river_edge_index[0], river_edge_index[1],
                                     mw_r, Er, 16 * ER_PT, N, 16, NC_R)
    aggr_c_p, aggr_r_p = _sc_aggregate(y_c, y_r, src_c, dst_c, mw_cp,
                                       src_r, dst_r, mw_rp)
    aggr_r = aggr_r_p.reshape(NPAIR, NPA, H2)
    aggr_c = aggr_c_p.reshape(NPAIR, NFP, H2)

    fused, pfln = _post(aggr_r, aggr_c, xg, fnx, r_upd_W, r_upd_b,
                        c_upd_W, c_upd_b, sf_W, sf_b, feat_emb, pf_g, pf_b)

    pfu = jnp.transpose(pfln.reshape(B, T, N, F, H), (0, 2, 3, 4, 1))

    xtp = jnp.transpose(x_global, (0, 1, 3, 2)).reshape(B * N, T, H)
    xs = jnp.transpose(fused.reshape(B, T, N, H), (0, 2, 1, 3)) \
        .reshape(B * N, T, H)
    out = _final(xtp, xs, conv_W, conv_b, fu_W1, fu_b1, fu_ln_g, fu_ln_b,
                 fu_W2, fu_b2)
    x_out = jnp.transpose(out.reshape(B, N, T, H), (0, 1, 3, 2))
    return (x_out, pfu)


# speedup vs baseline: 18.2468x; 1.0625x over previous
import functools

import jax
import jax.numpy as jnp
import numpy as np
from jax import lax
from jax.experimental import pallas as pl
from jax.experimental.pallas import tpu as pltpu
from jax.experimental.pallas import tpu_sc as plsc

B, N, H, T, F = 2, 1000, 64, 16, 8
Er, Ec = 8000, 128000
BT = B * T
NF = N * F


def _pe_const():
    pe = np.zeros((T, H), np.float32)
    pos = np.arange(T, dtype=np.float32)[:, None]
    div = np.exp(np.arange(0, H, 2, dtype=np.float32) * (-np.log(10000.0) / H))
    pe[:, 0::2] = np.sin(pos * div)
    pe[:, 1::2] = np.cos(pos * div)
    return jnp.asarray(pe)


def _silu(x):
    return x * jax.nn.sigmoid(x)


def _lrelu(x):
    return jnp.where(x >= 0, x, 0.01 * x)


def _edge_w_body(attr4_ref, cew_ref, rencW_ref, scal_ref, mwr_ref, mwc_ref):
    w = rencW_ref[...]
    attr4 = attr4_ref[...]
    ew = (attr4 * w[:, :1]).sum(0, keepdims=True) + scal_ref[0, :1]
    mwr_ref[...] = jnp.clip(scal_ref[1, :1] * ew, 0.0, 1.0)
    mwc_ref[...] = jnp.clip(scal_ref[2, 0] * cew_ref[...], 0.0, 1.0)


def _edge_weights(river_edge_attr, causal_edge_weight, r_enc_W, r_enc_b,
                  r_gate, c_gate):
    attr4 = river_edge_attr.T
    cew = causal_edge_weight.reshape(Ec // 128, 128)
    scal = jnp.stack([
        jnp.broadcast_to(r_enc_b[0], (128,)),
        jnp.broadcast_to(jax.nn.sigmoid(r_gate[0]), (128,)),
        jnp.broadcast_to(jax.nn.sigmoid(c_gate[0]), (128,)),
    ])
    rencW = jnp.broadcast_to(r_enc_W[0][:, None], (4, 128))
    mwr, mwc = pl.pallas_call(
        _edge_w_body,
        out_shape=(jax.ShapeDtypeStruct((1, Er), jnp.float32),
                   jax.ShapeDtypeStruct((Ec // 128, 128), jnp.float32)),
    )(attr4, cew, rencW, scal)
    return mwr[0], mwc.reshape(Ec)


def _prep_body(xg_ref, fnx_ref, rW_ref, rb_ref, cW_ref, cb_ref, emb_ref,
               yr_ref, yc_ref):
    f32 = jnp.float32
    ya = jnp.dot(xg_ref[0], rW_ref[...], preferred_element_type=f32) \
        + rb_ref[...]
    yb = jnp.dot(xg_ref[1], rW_ref[...], preferred_element_type=f32) \
        + rb_ref[...]
    yr_ref[0] = jnp.concatenate([ya, yb], axis=-1)
    fna = (fnx_ref[0].reshape(N, F, H) + emb_ref[...][None]).reshape(NF, H)
    fnb = (fnx_ref[1].reshape(N, F, H) + emb_ref[...][None]).reshape(NF, H)
    ca = jnp.dot(fna, cW_ref[...], preferred_element_type=f32) + cb_ref[...]
    cb = jnp.dot(fnb, cW_ref[...], preferred_element_type=f32) + cb_ref[...]
    yc_ref[0] = jnp.concatenate([ca, cb], axis=-1)


def _prep(xg, fnx, r_lin_W, r_lin_b, c_lin_W, c_lin_b, feat_emb):
    return pl.pallas_call(
        _prep_body,
        grid=(BT // 2,),
        in_specs=[
            pl.BlockSpec((2, N, H), lambda p: (p, 0, 0)),
            pl.BlockSpec((2, NF, H), lambda p: (p, 0, 0)),
            pl.BlockSpec((H, H), lambda p: (0, 0)),
            pl.BlockSpec((1, H), lambda p: (0, 0)),
            pl.BlockSpec((H, H), lambda p: (0, 0)),
            pl.BlockSpec((1, H), lambda p: (0, 0)),
            pl.BlockSpec((F, H), lambda p: (0, 0)),
        ],
        out_specs=[
            pl.BlockSpec((1, N, 2 * H), lambda p: (p, 0, 0)),
            pl.BlockSpec((1, NF, 2 * H), lambda p: (p, 0, 0)),
        ],
        out_shape=[jax.ShapeDtypeStruct((BT // 2, N, 2 * H), jnp.float32),
                   jax.ShapeDtypeStruct((BT // 2, NF, 2 * H), jnp.float32)],
    )(xg, fnx, r_lin_W.T, r_lin_b[None], c_lin_W.T, c_lin_b[None], feat_emb)


def _post_body(ar_ref, ac_ref, xg_ref, fnx_ref, rA_ref, rB_ref, rb_ref,
               cA_ref, cB_ref, cb_ref, sA_ref, sB_ref, sb_ref, emb_ref,
               png_ref, pnb_ref, fused_ref, pfln_ref):
    f32 = jnp.float32
    for q in range(2):
        ar = ar_ref[0][:N, q * H:(q + 1) * H]
        ac = ac_ref[0][:NF, q * H:(q + 1) * H]
        w = _lrelu(jnp.dot(ar, rA_ref[...], preferred_element_type=f32)
                   + jnp.dot(xg_ref[q], rB_ref[...],
                             preferred_element_type=f32)
                   + rb_ref[...])
        fn = (fnx_ref[q].reshape(N, F, H) + emb_ref[...][None]).reshape(NF, H)
        fu = _lrelu(jnp.dot(ac, cA_ref[...], preferred_element_type=f32)
                    + jnp.dot(fn, cB_ref[...], preferred_element_type=f32)
                    + cb_ref[...])
        fu4 = fu.reshape(N, F, H)
        pooled = fu4.mean(1)
        fused_ref[q] = _silu(jnp.dot(w, sA_ref[...],
                                     preferred_element_type=f32)
                             + jnp.dot(pooled, sB_ref[...],
                                       preferred_element_type=f32)
                             + sb_ref[...])
        m = fu.mean(-1, keepdims=True)
        v = ((fu - m) ** 2).mean(-1, keepdims=True)
        pfln_ref[q] = png_ref[...] * (fu - m) * jax.lax.rsqrt(v + 1e-5) \
            + pnb_ref[...]


def _post(aggr_r, aggr_c, xg, fnx, r_upd_W, r_upd_b, c_upd_W, c_upd_b,
          sf_W, sf_b, feat_emb, pf_g, pf_b):
    return pl.pallas_call(
        _post_body,
        grid=(BT // 2,),
        in_specs=[
            pl.BlockSpec((1, NPA, 2 * H), lambda p: (p, 0, 0)),
            pl.BlockSpec((1, NFP, 2 * H), lambda p: (p, 0, 0)),
            pl.BlockSpec((2, N, H), lambda p: (p, 0, 0)),
            pl.BlockSpec((2, NF, H), lambda p: (p, 0, 0)),
        ] + [pl.BlockSpec(None, lambda p: (0, 0))] * 12,
        out_specs=[
            pl.BlockSpec((2, N, H), lambda p: (p, 0, 0)),
            pl.BlockSpec((2, NF, H), lambda p: (p, 0, 0)),
        ],
        out_shape=[jax.ShapeDtypeStruct((BT, N, H), jnp.float32),
                   jax.ShapeDtypeStruct((BT, NF, H), jnp.float32)],
    )(aggr_r, aggr_c, xg, fnx,
      r_upd_W[:, :H].T, r_upd_W[:, H:].T, r_upd_b[None],
      c_upd_W[:, :H].T, c_upd_W[:, H:].T, c_upd_b[None],
      sf_W[:, :H].T, sf_W[:, H:].T, sf_b[None],
      feat_emb, pf_g[None], pf_b[None])


def _final_body(xt_ref, xs_ref, pe_ref, W0_ref, W1_ref, W2_ref, cb_ref,
                F1a_ref, F1b_ref, f1b_ref, lg_ref, lb_ref, F2_ref, f2b_ref,
                out_ref):
    f32 = jnp.float32
    C = xt_ref.shape[0]
    x = xt_ref[...] + pe_ref[...][None]
    zero = jnp.zeros((C, 1, H), f32)
    xm = jnp.concatenate([zero, x[:, :-1]], axis=1)
    xp = jnp.concatenate([x[:, 1:], zero], axis=1)
    x2 = x.reshape(C * T, H)
    xtemp = (jnp.dot(xm.reshape(C * T, H), W0_ref[...], preferred_element_type=f32)
             + jnp.dot(x2, W1_ref[...], preferred_element_type=f32)
             + jnp.dot(xp.reshape(C * T, H), W2_ref[...], preferred_element_type=f32)
             + cb_ref[...])
    h = (jnp.dot(xtemp, F1a_ref[...], preferred_element_type=f32)
         + jnp.dot(xs_ref[...].reshape(C * T, H), F1b_ref[...],
                   preferred_element_type=f32)
         + f1b_ref[...])
    m = h.mean(-1, keepdims=True)
    v = ((h - m) ** 2).mean(-1, keepdims=True)
    h = _silu(lg_ref[...] * (h - m) * jax.lax.rsqrt(v + 1e-5) + lb_ref[...])
    out = jnp.dot(h, F2_ref[...], preferred_element_type=f32) + f2b_ref[...]
    out_ref[...] = out.reshape(C, T, H)


def _final(xtp, xs, conv_W, conv_b, fu_W1, fu_b1, fu_ln_g, fu_ln_b,
           fu_W2, fu_b2):
    CH = 500
    return pl.pallas_call(
        _final_body,
        grid=(B * N // CH,),
        in_specs=[
            pl.BlockSpec((CH, T, H), lambda i: (i, 0, 0)),
            pl.BlockSpec((CH, T, H), lambda i: (i, 0, 0)),
        ] + [pl.BlockSpec(None, lambda i: (0, 0))] * 12,
        out_specs=pl.BlockSpec((CH, T, H), lambda i: (i, 0, 0)),
        out_shape=jax.ShapeDtypeStruct((B * N, T, H), jnp.float32),
    )(xtp, xs, _pe_const(),
      conv_W[:, :, 0].T, conv_W[:, :, 1].T, conv_W[:, :, 2].T, conv_b[None],
      fu_W1[:, :H].T, fu_W1[:, H:].T, fu_b1[None],
      fu_ln_g[None], fu_ln_b[None], fu_W2.T, fu_b2[None])


EC_PT = 8192
ER_PT = 512
CH = 128
NC_C = EC_PT // CH
NC_R = ER_PT // CH
NFP = 8000
NPA = 1000
H2 = 2 * H
NPAIR = BT // 2
PAIRS_PER_CORE = NPAIR // 2


def _sc_aggr_body(yc_hbm, yr_hbm, idxc_hbm, dstc_hbm, mwc_hbm,
                  idxr_hbm, dstr_hbm, mwr_hbm, outc_hbm, outr_hbm,
                  acc_c, acc_r, mwc, mwr, idxb, dstb, rows,
                  gs0, gs1, gs2, ss0, ss1, ss2,
                  es0, es1, es2, ds0, ds1, ds2):
    cid = lax.axis_index("c")
    sid = lax.axis_index("s")
    gsem = [gs0, gs1, gs2]
    ssem = [ss0, ss1, ss2]
    esem = [es0, es1, es2]
    dsem = [ds0, ds1, ds2]

    pltpu.sync_copy(mwc_hbm.at[sid], mwc)
    pltpu.sync_copy(mwr_hbm.at[sid], mwr)

    bufs = [rows.at[pl.ds(q * CH, CH)] for q in range(3)]

    def _scale_chunk(mw_ref, rbuf, j):
        def _g(g, _):
            mvec = mw_ref[j, pl.ds(g * 16, 16)]
            e0 = g * 16
            for k in range(16):
                m = mvec[k]
                for h in range(H2 // 16):
                    sl = pl.ds(h * 16, 16)
                    rbuf[e0 + k, sl] = rbuf[e0 + k, sl] * m
            return _
        lax.fori_loop(0, CH // 16, _g, None)

    def _graph_pass(y_hbm, idx_hbm, dst_hbm, mw_res, acc, nc, ib0, db0):
        def _step(j, r):
            rn, rp = (r + 1) % 3, (r + 2) % 3
            s_wait = (isinstance(j, int) and j >= 2) or not isinstance(j, int)
            if s_wait:
                @pl.when(j >= 2)
                def _():
                    pltpu.make_async_copy(bufs[rn], acc.at[dstb.at[rn]],
                                          ssem[rn]).wait()
            @pl.when(j + 1 < nc)
            def _():
                pltpu.async_copy(dst_hbm.at[db0 + j + 1], dstb.at[rn],
                                 dsem[rn])
            @pl.when(j + 2 < nc)
            def _():
                pltpu.async_copy(idx_hbm.at[ib0 + j + 2], idxb.at[rp],
                                 esem[rp])
            @pl.when(j + 1 < nc)
            def _():
                pltpu.make_async_copy(idx_hbm.at[ib0 + j + 1], idxb.at[rn],
                                      esem[rn]).wait()
                pltpu.async_copy(y_hbm.at[idxb.at[rn]], bufs[rn], gsem[rn])
            pltpu.make_async_copy(y_hbm.at[idxb.at[r]], bufs[r],
                                  gsem[r]).wait()
            _scale_chunk(mw_res, bufs[r], j)
            pltpu.make_async_copy(dst_hbm.at[db0 + j], dstb.at[r],
                                  dsem[r]).wait()
            pltpu.async_copy(bufs[r], acc.at[dstb.at[r]], ssem[r], add=True)

        pltpu.async_copy(idx_hbm.at[ib0], idxb.at[0], esem[0])
        pltpu.async_copy(idx_hbm.at[ib0 + 1], idxb.at[1], esem[1])
        pltpu.async_copy(dst_hbm.at[db0], dstb.at[0], dsem[0])
        pltpu.make_async_copy(idx_hbm.at[ib0], idxb.at[0], esem[0]).wait()
        pltpu.async_copy(y_hbm.at[idxb.at[0]], bufs[0], gsem[0])
        _step(0, 0)

        def _loop(i, _):
            for k in range(3):
                _step(1 + 3 * i + k, (1 + k) % 3)
            return _
        lax.fori_loop(0, (nc - 1) // 3, _loop, None)

        for jj in (nc - 2, nc - 1):
            r = jj % 3
            pltpu.make_async_copy(bufs[r], acc.at[dstb.at[r]],
                                  ssem[r]).wait()

    def _pair_step(p_local, _):
        p_glob = cid * PAIRS_PER_CORE + p_local

        def _zero_row(r, _):
            for h in range(H2 // 16):
                rows[r, pl.ds(h * 16, 16)] = jnp.zeros((16,), jnp.float32)
            return _
        lax.fori_loop(0, CH, _zero_row, None)

        @pl.when(sid < 15)
        def _():
            for q in range(4):
                pltpu.sync_copy(bufs[0],
                                acc_c.at[pl.ds(sid * 512 + q * CH, CH)])
            pltpu.sync_copy(bufs[0].at[pl.ds(0, 64)],
                            acc_r.at[pl.ds(sid * 64, 64)])

        @pl.when(sid == 15)
        def _():
            for q in range(2):
                pltpu.sync_copy(bufs[0], acc_c.at[pl.ds(7680 + q * CH, CH)])
            pltpu.sync_copy(bufs[0].at[pl.ds(0, 64)],
                            acc_c.at[pl.ds(7936, 64)])
            pltpu.sync_copy(bufs[0].at[pl.ds(0, 40)],
                            acc_r.at[pl.ds(960, 40)])

        plsc.subcore_barrier()

        _graph_pass(yc_hbm, idxc_hbm, dstc_hbm, mwc, acc_c, NC_C,
                    (p_glob * 16 + sid) * NC_C, sid * NC_C)
        _graph_pass(yr_hbm, idxr_hbm, dstr_hbm, mwr, acc_r, NC_R,
                    (p_glob * 16 + sid) * NC_R, sid * NC_R)

        plsc.subcore_barrier()

        @pl.when(sid < 15)
        def _():
            for q in range(4):
                off = sid * 512 + q * CH
                pltpu.sync_copy(acc_c.at[pl.ds(off, CH)],
                                outc_hbm.at[pl.ds(p_glob * NFP + off, CH)])
            pltpu.sync_copy(
                acc_r.at[pl.ds(sid * 64, 64)],
                outr_hbm.at[pl.ds(p_glob * NPA + sid * 64, 64)])

        @pl.when(sid == 15)
        def _():
            for q in range(2):
                off = 7680 + q * CH
                pltpu.sync_copy(acc_c.at[pl.ds(off, CH)],
                                outc_hbm.at[pl.ds(p_glob * NFP + off, CH)])
            pltpu.sync_copy(acc_c.at[pl.ds(7936, 64)],
                            outc_hbm.at[pl.ds(p_glob * NFP + 7936, 64)])
            pltpu.sync_copy(acc_r.at[pl.ds(960, 40)],
                            outr_hbm.at[pl.ds(p_glob * NPA + 960, 40)])
        return _

    lax.fori_loop(0, PAIRS_PER_CORE, _pair_step, None)


def _sc_aggregate(y_c, y_r, idx_c, dst_c, mw_c, idx_r, dst_r, mw_r):
    mesh = plsc.VectorSubcoreMesh(core_axis_name="c", subcore_axis_name="s",
                                  num_cores=2, num_subcores=16)
    f32, i32 = jnp.float32, jnp.int32
    kern = pl.kernel(
        _sc_aggr_body,
        out_type=(jax.ShapeDtypeStruct((NPAIR * NFP, H2), f32),
                  jax.ShapeDtypeStruct((NPAIR * NPA, H2), f32)),
        mesh=mesh,
        scratch_types=[
            pltpu.VMEM_SHARED((NFP, H2), f32),
            pltpu.VMEM_SHARED((NPA, H2), f32),
            pltpu.VMEM((NC_C, CH), f32),
            pltpu.VMEM((NC_R, CH), f32),
            pltpu.VMEM((3, CH), i32),
            pltpu.VMEM((3, CH), i32),
            pltpu.VMEM((3 * CH, H2), f32),
        ] + [pltpu.SemaphoreType.DMA] * 12,
    )
    return kern(y_c.reshape(NPAIR * NF, H2), y_r.reshape(NPAIR * N, H2),
                idx_c, dst_c, mw_c, idx_r, dst_r, mw_r)


def _pad_edges(src, dst, mw, e_real, e_pad, nseg, ntile, nchunk):
    pad = e_pad - e_real
    fill = jnp.arange(pad, dtype=jnp.int32) % nseg
    srcp = jnp.concatenate([src.astype(jnp.int32), fill])
    dstp = jnp.concatenate([dst.astype(jnp.int32), fill])
    mwp = jnp.concatenate([mw, jnp.zeros((pad,), jnp.float32)])
    rs = lambda a: a.reshape(ntile, nchunk, CH)
    return rs(srcp), rs(dstp), rs(mwp)


def kernel(x_global, per_feature_x, river_edge_index, river_edge_attr,
           causal_edge_index, causal_edge_weight, r_lin_W, r_lin_b, r_upd_W,
           r_upd_b, r_enc_W, r_enc_b, r_gate, c_lin_W, c_lin_b, c_upd_W,
           c_upd_b, c_gate, conv_W, conv_b, sf_W, sf_b, feat_emb, pf_g, pf_b,
           fu_W1, fu_b1, fu_ln_g, fu_ln_b, fu_W2, fu_b2):
    xg = jnp.transpose(x_global, (0, 3, 1, 2)).reshape(BT, N, H)
    fnx = jnp.transpose(per_feature_x, (0, 4, 1, 2, 3)).reshape(BT, NF, H)

    mw_r, mw_c = _edge_weights(river_edge_attr, causal_edge_weight,
                               r_enc_W, r_enc_b, r_gate, c_gate)
    y_r, y_c = _prep(xg, fnx, r_lin_W, r_lin_b, c_lin_W, c_lin_b, feat_emb)

    src_c, dst_c, mw_cp = _pad_edges(causal_edge_index[0],
                                     causal_edge_index[1], mw_c,
                                     Ec, 16 * EC_PT, NF, 16, NC_C)
    src_r, dst_r, mw_rp = _pad_edges(river_edge_index[0], river_edge_index[1],
                                     mw_r, Er, 16 * ER_PT, N, 16, NC_R)
    pair_off_c = (jnp.arange(NPAIR, dtype=jnp.int32) * NF)[:, None, None, None]
    idx_c = (src_c[None] + pair_off_c).reshape(NPAIR * 16 * NC_C, CH)
    pair_off_r = (jnp.arange(NPAIR, dtype=jnp.int32) * N)[:, None, None, None]
    idx_r = (src_r[None] + pair_off_r).reshape(NPAIR * 16 * NC_R, CH)
    aggr_c_p, aggr_r_p = _sc_aggregate(y_c, y_r,
                                       idx_c, dst_c.reshape(16 * NC_C, CH),
                                       mw_cp,
                                       idx_r, dst_r.reshape(16 * NC_R, CH),
                                       mw_rp)
    aggr_r = aggr_r_p.reshape(NPAIR, NPA, H2)
    aggr_c = aggr_c_p.reshape(NPAIR, NFP, H2)

    fused, pfln = _post(aggr_r, aggr_c, xg, fnx, r_upd_W, r_upd_b,
                        c_upd_W, c_upd_b, sf_W, sf_b, feat_emb, pf_g, pf_b)

    pfu = jnp.transpose(pfln.reshape(B, T, N, F, H), (0, 2, 3, 4, 1))

    xtp = jnp.transpose(x_global, (0, 1, 3, 2)).reshape(B * N, T, H)
    xs = jnp.transpose(fused.reshape(B, T, N, H), (0, 2, 1, 3)) \
        .reshape(B * N, T, H)
    out = _final(xtp, xs, conv_W, conv_b, fu_W1, fu_b1, fu_ln_g, fu_ln_b,
                 fu_W2, fu_b2)
    x_out = jnp.transpose(out.reshape(B, N, T, H), (0, 1, 3, 2))
    return (x_out, pfu)
